# baseline (speedup 1.0000x reference)
import functools

import jax
import jax.numpy as jnp
from jax import lax
from jax.experimental import pallas as pl
from jax.experimental.pallas import tpu as pltpu
from jax.experimental.pallas import tpu_sc as plsc

B, S, D, E, F, K = 1, 2048, 768, 8, 2048, 2
N = S * K
T = 128
P = N + E * T
NT = P // T
NW = 32
PPW = N // NW
SPW = P // NW
TPW = S // NW

_mesh = functools.partial(
    plsc.VectorSubcoreMesh, core_axis_name="c", subcore_axis_name="s",
    num_cores=2, num_subcores=16)


def _wid():
    return lax.axis_index("s") * 2 + lax.axis_index("c")


def _gate_body(x_ref, gw_ref, gb_ref, e_ref, w_ref):
    x = x_ref[...]
    logits = jnp.dot(x, gw_ref[...], preferred_element_type=jnp.float32)
    logits = logits + gb_ref[...]
    iota = lax.broadcasted_iota(jnp.int32, (S, E), 1)
    m0 = jnp.max(logits, axis=1, keepdims=True)
    e0 = jnp.min(jnp.where(logits == m0, iota, E), axis=1, keepdims=True)
    mask0 = iota == e0
    l1m = jnp.where(mask0, jnp.float32(-1e30), logits)
    m1 = jnp.max(l1m, axis=1, keepdims=True)
    e1 = jnp.min(jnp.where(l1m == m1, iota, E), axis=1, keepdims=True)
    a = jnp.exp(m1 - m0)
    w0 = 1.0 / (1.0 + a)
    e_ref[...] = jnp.concatenate([e0, e1], axis=1)
    w_ref[...] = jnp.concatenate([w0, 1.0 - w0], axis=1)


def _gating(x2d, gate_w, gate_b):
    return pl.pallas_call(
        _gate_body,
        out_shape=(
            jax.ShapeDtypeStruct((S, K), jnp.int32),
            jax.ShapeDtypeStruct((S, K), jnp.float32),
        ),
    )(x2d, gate_w, gate_b.reshape(1, E))


def _lane(vec, e):
    iota16 = lax.iota(jnp.int32, 16)
    return jnp.max(jnp.where(iota16 == e, vec, jnp.int32(-2147483647)))


def _route_body(eflat, wflat, tsort, wsort, invpos, gtile, nact,
                eb, wb, posb, tokb, s16):
    wid = _wid()
    iota16 = lax.iota(jnp.int32, 16)
    pltpu.sync_copy(eflat, eb)
    pltpu.sync_copy(wflat.at[pl.ds(wid * PPW, PPW)], wb)
    myc0 = wid * (PPW // 16)

    def hist_body(c, carry):
        cnt, pre = carry
        pre = jnp.where(c == myc0, cnt, pre)
        ev = eb[pl.ds(c * 16, 16)]
        upd = jnp.zeros((16,), jnp.int32)
        for e in range(E):
            pc = plsc.all_reduce_population_count(ev == e)
            upd = upd + jnp.where(iota16 == e, pc, 0)
        return cnt + upd, pre

    zero16 = jnp.zeros((16,), jnp.int32)
    cnt, pre = lax.fori_loop(0, N // 16, hist_body, (zero16, zero16))

    padded = ((cnt + (T - 1)) // T) * T
    incl = plsc.cumsum(padded)
    base = (incl - padded) + pre

    for c2 in range(PPW // 16):
        ev = eb[pl.ds((myc0 + c2) * 16, 16)]
        pos = jnp.zeros((16,), jnp.int32)
        delta = jnp.zeros((16,), jnp.int32)
        for e in range(E):
            m = ev == e
            r = plsc.cumsum(jnp.where(m, 1, 0))
            pos = jnp.where(m, _lane(base, e) + r - 1, pos)
            pc = plsc.all_reduce_population_count(m)
            delta = delta + jnp.where(iota16 == e, pc, 0)
        base = base + delta
        posb[pl.ds(c2 * 16, 16)] = pos
        p_glob = wid * PPW + c2 * 16 + iota16
        tokb[pl.ds(c2 * 16, 16)] = p_glob - jnp.where(p_glob >= S, S, 0)

    pltpu.sync_copy(posb, invpos.at[pl.ds(wid * PPW, PPW)])
    pltpu.sync_copy(tokb, tsort.at[posb])
    pltpu.sync_copy(wb, wsort.at[posb])

    @pl.when(wid == 0)
    def _():
        la = jnp.max(jnp.where((cnt > 0) & (iota16 < E), iota16, 0))
        total = jnp.max(incl)
        s16[...] = jnp.broadcast_to(total // T, (16,))
        pltpu.sync_copy(s16, nact)
        for j in range(3):
            tstart = (j * 16 + iota16) * T
            acc = jnp.zeros((16,), jnp.int32)
            for e in range(E):
                acc = acc + jnp.where(tstart >= _lane(incl, e), 1, 0)
            s16[...] = jnp.minimum(acc, la)
            pltpu.sync_copy(s16, gtile.at[pl.ds(j * 16, 16)])


def _route(eflat, wflat):
    return pl.kernel(
        _route_body,
        out_type=(
            jax.ShapeDtypeStruct((P,), jnp.int32),
            jax.ShapeDtypeStruct((P,), jnp.float32),
            jax.ShapeDtypeStruct((N,), jnp.int32),
            jax.ShapeDtypeStruct((48,), jnp.int32),
            jax.ShapeDtypeStruct((16,), jnp.int32),
        ),
        mesh=_mesh(),
        compiler_params=pltpu.CompilerParams(needs_layout_passes=False),
        scratch_types=[
            pltpu.VMEM((N,), jnp.int32),
            pltpu.VMEM((PPW,), jnp.float32),
            pltpu.VMEM((PPW,), jnp.int32),
            pltpu.VMEM((PPW,), jnp.int32),
            pltpu.VMEM((16,), jnp.int32),
        ],
    )(eflat, wflat)


def _gather_body(tsort, x2d, xs, idxa, idxb, rows, sem):
    wid = _wid()
    base = wid * SPW
    half = SPW // 2
    pltpu.sync_copy(tsort.at[pl.ds(base, half)], idxa)
    pltpu.sync_copy(tsort.at[pl.ds(base + half, half)], idxb)
    for j in range(half // 16):
        sl = pl.ds(j * 16, 16)
        idxa[sl] = jnp.clip(idxa[sl], 0, S - 1)
        idxb[sl] = jnp.clip(idxb[sl], 0, S - 1)
    pltpu.async_copy(x2d.at[idxa], rows, sem).wait()
    pltpu.sync_copy(rows, xs.at[pl.ds(base, half)])
    pltpu.async_copy(x2d.at[idxb], rows, sem).wait()
    pltpu.sync_copy(rows, xs.at[pl.ds(base + half, half)])


def _gather(tsort, x2d):
    return pl.kernel(
        _gather_body,
        out_type=jax.ShapeDtypeStruct((P, D), jnp.float32),
        mesh=_mesh(),
        compiler_params=pltpu.CompilerParams(needs_layout_passes=False),
        scratch_types=[
            pltpu.VMEM((SPW // 2,), jnp.int32),
            pltpu.VMEM((SPW // 2,), jnp.int32),
            pltpu.VMEM((SPW // 2, D), jnp.float32),
            pltpu.SemaphoreType.DMA,
        ],
    )(tsort, x2d)


def _gemm_body(scal, xs_ref, wsc_ref, w1_ref, b1_ref, w2_ref, b2_ref, out_ref):
    i = pl.program_id(0)

    @pl.when(i < scal[NT])
    def _():
        h = jnp.dot(xs_ref[...], w1_ref[0], preferred_element_type=jnp.float32)
        h = jnp.maximum(h + b1_ref[0], 0.0)
        o = jnp.dot(h, w2_ref[0], preferred_element_type=jnp.float32)
        out_ref[...] = (o + b2_ref[0]) * wsc_ref[...]


def _grouped_gemm(scal, xs, wsc, w1, b1, w2, b2):
    grid_spec = pltpu.PrefetchScalarGridSpec(
        num_scalar_prefetch=1,
        grid=(NT,),
        in_specs=[
            pl.BlockSpec((T, D), lambda i, s: (i, 0)),
            pl.BlockSpec((T, 1), lambda i, s: (i, 0)),
            pl.BlockSpec((1, D, F), lambda i, s: (s[i], 0, 0)),
            pl.BlockSpec((1, 1, F), lambda i, s: (s[i], 0, 0)),
            pl.BlockSpec((1, F, D), lambda i, s: (s[i], 0, 0)),
            pl.BlockSpec((1, 1, D), lambda i, s: (s[i], 0, 0)),
        ],
        out_specs=pl.BlockSpec((T, D), lambda i, s: (i, 0)),
    )
    return pl.pallas_call(
        _gemm_body,
        grid_spec=grid_spec,
        out_shape=jax.ShapeDtypeStruct((P, D), jnp.float32),
    )(scal, xs, wsc, w1, b1.reshape(E, 1, F), w2, b2.reshape(E, 1, D))


def _combine_body(contrib, invpos, y, idx0, idx1, ra, rb, sem):
    wid = _wid()
    base = wid * TPW
    pltpu.sync_copy(invpos.at[pl.ds(base, TPW)], idx0)
    pltpu.sync_copy(invpos.at[pl.ds(S + base, TPW)], idx1)
    pltpu.async_copy(contrib.at[idx0], ra, sem).wait()
    pltpu.async_copy(contrib.at[idx1], rb, sem).wait()

    def row_body(r, _):
        for u in range(D // 16):
            sl = pl.ds(u * 16, 16)
            ra[r, sl] = ra[r, sl] + rb[r, sl]
        return 0

    lax.fori_loop(0, TPW, row_body, 0)
    pltpu.sync_copy(ra, y.at[pl.ds(base, TPW)])


def _combine(contrib, invpos):
    return pl.kernel(
        _combine_body,
        out_type=jax.ShapeDtypeStruct((S, D), jnp.float32),
        mesh=_mesh(),
        compiler_params=pltpu.CompilerParams(needs_layout_passes=False),
        scratch_types=[
            pltpu.VMEM((TPW,), jnp.int32),
            pltpu.VMEM((TPW,), jnp.int32),
            pltpu.VMEM((TPW, D), jnp.float32),
            pltpu.VMEM((TPW, D), jnp.float32),
            pltpu.SemaphoreType.DMA,
        ],
    )(contrib, invpos)


def kernel(x, gate_w, gate_b, w1, b1, w2, b2):
    x2d = x.reshape(S, D)
    e_sk, w_sk = _gating(x2d, gate_w, gate_b)
    eflat = e_sk.T.reshape(N)
    wflat = w_sk.T.reshape(N)
    tsort, wsort, invpos, gtile, nact = _route(eflat, wflat)
    xs = _gather(tsort, x2d)
    scal = jnp.concatenate([gtile[:NT], nact[:1]])
    contrib = _grouped_gemm(scal, xs, wsort.reshape(P, 1), w1, b1, w2, b2)
    y = _combine(contrib, invpos)
    return y.reshape(B, S, D)


# speedup vs baseline: 1.3764x; 1.3764x over previous
import functools

import jax
import jax.numpy as jnp
from jax import lax
from jax.experimental import pallas as pl
from jax.experimental.pallas import tpu as pltpu
from jax.experimental.pallas import tpu_sc as plsc

B, S, D, E, F, K = 1, 2048, 768, 8, 2048, 2
N = S * K
T = 128
P = N + E * T
NT = P // T
NW = 32
PPW = N // NW
SPW = P // NW
TPW = S // NW

_mesh = functools.partial(
    plsc.VectorSubcoreMesh, core_axis_name="c", subcore_axis_name="s",
    num_cores=2, num_subcores=16)


def _wid():
    return lax.axis_index("s") * 2 + lax.axis_index("c")


def _gate_body(x_ref, gw_ref, gb_ref, e_ref, w_ref):
    x = x_ref[...]
    logits = jnp.dot(x, gw_ref[...], preferred_element_type=jnp.float32)
    logits = logits + gb_ref[...]
    iota = lax.broadcasted_iota(jnp.int32, (S, E), 1)
    m0 = jnp.max(logits, axis=1, keepdims=True)
    e0 = jnp.min(jnp.where(logits == m0, iota, E), axis=1, keepdims=True)
    mask0 = iota == e0
    l1m = jnp.where(mask0, jnp.float32(-1e30), logits)
    m1 = jnp.max(l1m, axis=1, keepdims=True)
    e1 = jnp.min(jnp.where(l1m == m1, iota, E), axis=1, keepdims=True)
    a = jnp.exp(m1 - m0)
    w0 = 1.0 / (1.0 + a)
    e_ref[...] = jnp.concatenate([e0, e1], axis=1)
    w_ref[...] = jnp.concatenate([w0, 1.0 - w0], axis=1)


def _gating(x2d, gate_w, gate_b):
    return pl.pallas_call(
        _gate_body,
        out_shape=(
            jax.ShapeDtypeStruct((S, K), jnp.int32),
            jax.ShapeDtypeStruct((S, K), jnp.float32),
        ),
    )(x2d, gate_w, gate_b.reshape(1, E))


def _lane(vec, e):
    iota16 = lax.iota(jnp.int32, 16)
    return jnp.max(jnp.where(iota16 == e, vec, jnp.int32(-2147483647)))


def _route_body(eflat, wflat, x2d, xs, wsort, invpos, gtile, nact,
                eb, wb, posa, posb, toka, tokb, rows, s16, sem):
    wid = _wid()
    iota16 = lax.iota(jnp.int32, 16)
    pltpu.sync_copy(eflat, eb)
    pltpu.sync_copy(wflat.at[pl.ds(wid * PPW, PPW)], wb)
    myc0 = wid * (PPW // 16)

    def hist_body(c, carry):
        cnt, pre = carry
        pre = jnp.where(c == myc0, cnt, pre)
        ev = eb[pl.ds(c * 16, 16)]
        upd = jnp.zeros((16,), jnp.int32)
        for e in range(E):
            pc = plsc.all_reduce_population_count(ev == e)
            upd = upd + jnp.where(iota16 == e, pc, 0)
        return cnt + upd, pre

    zero16 = jnp.zeros((16,), jnp.int32)
    cnt, pre = lax.fori_loop(0, N // 16, hist_body, (zero16, zero16))

    padded = ((cnt + (T - 1)) // T) * T
    incl = plsc.cumsum(padded)
    base = (incl - padded) + pre

    half = PPW // 2
    for c2 in range(PPW // 16):
        ev = eb[pl.ds((myc0 + c2) * 16, 16)]
        pos = jnp.zeros((16,), jnp.int32)
        delta = jnp.zeros((16,), jnp.int32)
        for e in range(E):
            m = ev == e
            r = plsc.cumsum(jnp.where(m, 1, 0))
            pos = jnp.where(m, _lane(base, e) + r - 1, pos)
            pc = plsc.all_reduce_population_count(m)
            delta = delta + jnp.where(iota16 == e, pc, 0)
        base = base + delta
        p_glob = wid * PPW + c2 * 16 + iota16
        tok = p_glob - jnp.where(p_glob >= S, S, 0)
        hi = c2 >= (PPW // 32)
        dst_pos, dst_tok = (posb, tokb) if hi else (posa, toka)
        off = (c2 - (PPW // 32)) * 16 if hi else c2 * 16
        dst_pos[pl.ds(off, 16)] = pos
        dst_tok[pl.ds(off, 16)] = tok
        s16[...] = pos
        pltpu.sync_copy(s16, invpos.at[pl.ds(wid * PPW + c2 * 16, 16)])

    pltpu.async_copy(x2d.at[toka], rows, sem).wait()
    pltpu.async_copy(rows, xs.at[posa], sem).wait()
    pltpu.async_copy(x2d.at[tokb], rows, sem).wait()
    pltpu.async_copy(rows, xs.at[posb], sem).wait()
    pltpu.async_copy(wb.at[pl.ds(0, half)], wsort.at[posa], sem).wait()
    pltpu.async_copy(wb.at[pl.ds(half, half)], wsort.at[posb], sem).wait()

    @pl.when(wid == 0)
    def _():
        la = jnp.max(jnp.where((cnt > 0) & (iota16 < E), iota16, 0))
        total = jnp.max(incl)
        s16[...] = jnp.broadcast_to(total // T, (16,))
        pltpu.sync_copy(s16, nact)
        for j in range(3):
            tstart = (j * 16 + iota16) * T
            acc = jnp.zeros((16,), jnp.int32)
            for e in range(E):
                acc = acc + jnp.where(tstart >= _lane(incl, e), 1, 0)
            s16[...] = jnp.minimum(acc, la)
            pltpu.sync_copy(s16, gtile.at[pl.ds(j * 16, 16)])


def _route(eflat, wflat, x2d):
    return pl.kernel(
        _route_body,
        out_type=(
            jax.ShapeDtypeStruct((P, D), jnp.float32),
            jax.ShapeDtypeStruct((P,), jnp.float32),
            jax.ShapeDtypeStruct((N,), jnp.int32),
            jax.ShapeDtypeStruct((48,), jnp.int32),
            jax.ShapeDtypeStruct((16,), jnp.int32),
        ),
        mesh=_mesh(),
        compiler_params=pltpu.CompilerParams(needs_layout_passes=False),
        scratch_types=[
            pltpu.VMEM((N,), jnp.int32),
            pltpu.VMEM((PPW,), jnp.float32),
            pltpu.VMEM((PPW // 2,), jnp.int32),
            pltpu.VMEM((PPW // 2,), jnp.int32),
            pltpu.VMEM((PPW // 2,), jnp.int32),
            pltpu.VMEM((PPW // 2,), jnp.int32),
            pltpu.VMEM((PPW // 2, D), jnp.float32),
            pltpu.VMEM((16,), jnp.int32),
            pltpu.SemaphoreType.DMA,
        ],
    )(eflat, wflat, x2d)


def _gemm_body(scal, xs_ref, wsc_ref, w1_ref, b1_ref, w2_ref, b2_ref, out_ref):
    i = pl.program_id(0)

    @pl.when(i < scal[NT])
    def _():
        h = jnp.dot(xs_ref[...], w1_ref[0], preferred_element_type=jnp.float32)
        h = jnp.maximum(h + b1_ref[0], 0.0)
        o = jnp.dot(h, w2_ref[0], preferred_element_type=jnp.float32)
        out_ref[...] = (o + b2_ref[0]) * wsc_ref[...]


def _grouped_gemm(scal, xs, wsc, w1, b1, w2, b2):
    grid_spec = pltpu.PrefetchScalarGridSpec(
        num_scalar_prefetch=1,
        grid=(NT,),
        in_specs=[
            pl.BlockSpec((T, D), lambda i, s: (i, 0)),
            pl.BlockSpec((T, 1), lambda i, s: (i, 0)),
            pl.BlockSpec((1, D, F), lambda i, s: (s[i], 0, 0)),
            pl.BlockSpec((1, 1, F), lambda i, s: (s[i], 0, 0)),
            pl.BlockSpec((1, F, D), lambda i, s: (s[i], 0, 0)),
            pl.BlockSpec((1, 1, D), lambda i, s: (s[i], 0, 0)),
        ],
        out_specs=pl.BlockSpec((T, D), lambda i, s: (i, 0)),
    )
    return pl.pallas_call(
        _gemm_body,
        grid_spec=grid_spec,
        out_shape=jax.ShapeDtypeStruct((P, D), jnp.float32),
    )(scal, xs, wsc, w1, b1.reshape(E, 1, F), w2, b2.reshape(E, 1, D))


def _combine_body(contrib, invpos, y, idx0, idx1, ra, rb, sem):
    wid = _wid()
    base = wid * TPW
    pltpu.sync_copy(invpos.at[pl.ds(base, TPW)], idx0)
    pltpu.sync_copy(invpos.at[pl.ds(S + base, TPW)], idx1)
    pltpu.async_copy(contrib.at[idx0], ra, sem).wait()
    pltpu.async_copy(contrib.at[idx1], rb, sem).wait()

    def row_body(r, _):
        for u in range(D // 16):
            sl = pl.ds(u * 16, 16)
            ra[r, sl] = ra[r, sl] + rb[r, sl]
        return 0

    lax.fori_loop(0, TPW, row_body, 0)
    pltpu.sync_copy(ra, y.at[pl.ds(base, TPW)])


def _combine(contrib, invpos):
    return pl.kernel(
        _combine_body,
        out_type=jax.ShapeDtypeStruct((S, D), jnp.float32),
        mesh=_mesh(),
        compiler_params=pltpu.CompilerParams(needs_layout_passes=False),
        scratch_types=[
            pltpu.VMEM((TPW,), jnp.int32),
            pltpu.VMEM((TPW,), jnp.int32),
            pltpu.VMEM((TPW, D), jnp.float32),
            pltpu.VMEM((TPW, D), jnp.float32),
            pltpu.SemaphoreType.DMA,
        ],
    )(contrib, invpos)


def kernel(x, gate_w, gate_b, w1, b1, w2, b2):
    x2d = x.reshape(S, D)
    e_sk, w_sk = _gating(x2d, gate_w, gate_b)
    eflat = e_sk.T.reshape(N)
    wflat = w_sk.T.reshape(N)
    xs, wsort, invpos, gtile, nact = _route(eflat, wflat, x2d)
    scal = jnp.concatenate([gtile[:NT], nact[:1]])
    contrib = _grouped_gemm(scal, xs, wsort.reshape(P, 1), w1, b1, w2, b2)
    y = _combine(contrib, invpos)
    return y.reshape(B, S, D)
